# baseline (speedup 1.0000x reference)
import functools

import jax
import jax.numpy as jnp
from jax import lax
from jax.experimental import pallas as pl
from jax.experimental.pallas import tpu as pltpu
from jax.experimental.pallas import tpu_sc as plsc

N = 10000
NP = 10240
E = 320000
D = 128
R = 9
NB = 6

NW = 32
NPT = NP // NW
CHUNK = 8000
NCHUNK = E // CHUNK
GB = 128
BN = 512
NBLK = NP // BN


def _w_body(comp_ref, basis_ref, out_ref):
    r = pl.program_id(0)
    onehot = (lax.broadcasted_iota(jnp.int32, (R, 1), 0) == r).astype(jnp.float32)
    cw = jnp.sum(comp_ref[...] * onehot, axis=0)
    out_ref[0] = jnp.sum(basis_ref[...] * cw[:, None, None], axis=0)


def _assemble_w(comp, basis, d_in, d_out):
    return pl.pallas_call(
        _w_body,
        grid=(R,),
        in_specs=[
            pl.BlockSpec((R, NB), lambda r: (0, 0)),
            pl.BlockSpec((NB, d_in, d_out), lambda r: (0, 0, 0)),
        ],
        out_specs=pl.BlockSpec((1, d_in, d_out), lambda r: (r, 0, 0)),
        out_shape=jax.ShapeDtypeStruct((R, d_in, d_out), jnp.float32),
    )(comp, basis)


def _h_body(x_ref, w_ref, out_ref):
    out_ref[0] = jnp.dot(x_ref[...], w_ref[0],
                         preferred_element_type=jnp.float32)


def _relation_matmul(x, w):
    d_in, d_out = w.shape[1], w.shape[2]
    return pl.pallas_call(
        _h_body,
        grid=(R, NBLK),
        in_specs=[
            pl.BlockSpec((BN, d_in), lambda r, i: (i, 0)),
            pl.BlockSpec((1, d_in, d_out), lambda r, i: (r, 0, 0)),
        ],
        out_specs=pl.BlockSpec((1, BN, d_out), lambda r, i: (r, i, 0)),
        out_shape=jax.ShapeDtypeStruct((R, NP, d_out), jnp.float32),
    )(x, w)


def _ln(h, g, b):
    mu = jnp.mean(h, axis=-1, keepdims=True)
    var = jnp.mean((h - mu) ** 2, axis=-1, keepdims=True)
    return (h - mu) / jnp.sqrt(var + 1e-5) * g + b


def _mid_body(agg_ref, x_ref, root_ref, bias_ref, g_ref, b_ref, out_ref):
    a = agg_ref[...]
    a = jnp.where(jnp.isfinite(a), a, 0.0)
    h = a + jnp.dot(x_ref[...], root_ref[...],
                    preferred_element_type=jnp.float32) + bias_ref[0][None, :]
    h = _ln(h, g_ref[0][None, :], b_ref[0][None, :])
    out_ref[...] = jnp.maximum(h, 0.0)


def _mid_layer(agg, x, root, bias, g, b):
    vec = lambda i: (i, 0)
    return pl.pallas_call(
        _mid_body,
        grid=(NBLK,),
        in_specs=[
            pl.BlockSpec((BN, D), vec),
            pl.BlockSpec((BN, D), vec),
            pl.BlockSpec((D, D), lambda i: (0, 0)),
            pl.BlockSpec((1, D), lambda i: (0, 0)),
            pl.BlockSpec((1, D), lambda i: (0, 0)),
            pl.BlockSpec((1, D), lambda i: (0, 0)),
        ],
        out_specs=pl.BlockSpec((BN, D), vec),
        out_shape=jax.ShapeDtypeStruct((NP, D), jnp.float32),
    )(agg, x, root, bias.reshape(1, D), g.reshape(1, D), b.reshape(1, D))


def _tail_body(agg_ref, aggb_ref, x1_ref, root_ref, bias_ref, g_ref, b_ref,
               w1_ref, b1_ref, w2_ref, b2_ref, w3_ref, b3_ref, out_ref):
    x1 = x1_ref[...]
    h = agg_ref[...] + aggb_ref[...] + jnp.dot(
        x1, root_ref[...], preferred_element_type=jnp.float32)
    h = h + bias_ref[0][None, :]
    h = _ln(h, g_ref[0][None, :], b_ref[0][None, :])
    h = jnp.maximum(h, 0.0)
    h = h + x1 * 0.5
    h = jax.nn.gelu(jnp.dot(h, w1_ref[...],
                            preferred_element_type=jnp.float32)
                    + b1_ref[0][None, :])
    h = jax.nn.gelu(jnp.dot(h, w2_ref[...],
                            preferred_element_type=jnp.float32)
                    + b2_ref[0][None, :])
    o = jnp.dot(h, w3_ref[...], preferred_element_type=jnp.float32) \
        + b3_ref[0][None, :]
    m = jnp.max(o, axis=-1, keepdims=True)
    lse = m + jnp.log(jnp.sum(jnp.exp(o - m), axis=-1, keepdims=True))
    out_ref[...] = o - lse


def _tail_layer(agg, aggb, x1, root, bias, g, b, w1, b1, w2, b2, w3, b3):
    vec = lambda i: (i, 0)
    fix = lambda i: (0, 0)
    dh2 = D // 2
    do = 64
    return pl.pallas_call(
        _tail_body,
        grid=(NBLK,),
        in_specs=[
            pl.BlockSpec((BN, D), vec),
            pl.BlockSpec((BN, D), vec),
            pl.BlockSpec((BN, D), vec),
            pl.BlockSpec((D, D), fix),
            pl.BlockSpec((1, D), fix),
            pl.BlockSpec((1, D), fix),
            pl.BlockSpec((1, D), fix),
            pl.BlockSpec((D, D), fix),
            pl.BlockSpec((1, D), fix),
            pl.BlockSpec((D, dh2), fix),
            pl.BlockSpec((1, dh2), fix),
            pl.BlockSpec((dh2, do), fix),
            pl.BlockSpec((1, do), fix),
        ],
        out_specs=pl.BlockSpec((BN, do), vec),
        out_shape=jax.ShapeDtypeStruct((NP, do), jnp.float32),
    )(agg, aggb, x1, root, bias.reshape(1, D), g.reshape(1, D),
      b.reshape(1, D), w1, b1.reshape(1, D), w2, b2.reshape(1, dh2),
      w3, b3.reshape(1, do))


def _make_sc_agg(is_max):
    mesh = plsc.VectorSubcoreMesh(core_axis_name="c", subcore_axis_name="s")
    init_val = -jnp.inf if is_max else 0.0

    @functools.partial(
        pl.kernel,
        mesh=mesh,
        compiler_params=pltpu.CompilerParams(needs_layout_passes=False),
        out_type=jax.ShapeDtypeStruct((NP * D,), jnp.float32),
        scratch_types=[
            pltpu.VMEM((CHUNK,), jnp.int32),
            pltpu.VMEM((CHUNK,), jnp.int32),
            pltpu.VMEM((CHUNK,), jnp.int32),
            pltpu.VMEM((CHUNK + GB,), jnp.int32),
            pltpu.VMEM((CHUNK + GB,), jnp.int32),
            pltpu.VMEM((GB, D), jnp.float32),
            pltpu.VMEM(((NPT + 1) * D,), jnp.float32),
            pltpu.VMEM((CHUNK + GB,), jnp.int32),
            pltpu.SemaphoreType.DMA,
        ],
    )
    def agg_kernel(h_hbm, src_hbm, dst_hbm, et_hbm, out_hbm,
                   srcv, dstv, etv, flatv, dstlv, rows, acc, posv, sem):
        wid = lax.axis_index("s") * 2 + lax.axis_index("c")
        lo = wid * NPT

        fill = jnp.full((16,), init_val, jnp.float32)
        arange = jnp.arange(16, dtype=jnp.int32)
        ones = jnp.ones((16,), jnp.int32)
        zeros = jnp.zeros((16,), jnp.int32)
        trash = jnp.full((16,), NPT, jnp.int32)

        def init_row(i, _):
            acc[pl.ds(i * 16, 16)] = fill
            return 0

        lax.fori_loop(0, (NPT + 1) * D // 16, init_row, 0)

        def chunk_body(c, _):
            base = c * CHUNK
            pltpu.sync_copy(src_hbm.at[pl.ds(base, CHUNK)], srcv)
            pltpu.sync_copy(dst_hbm.at[pl.ds(base, CHUNK)], dstv)
            pltpu.sync_copy(et_hbm.at[pl.ds(base, CHUNK)], etv)

            def select(i, off):
                sl = pl.ds(i * 16, 16)
                d = dstv[sl]
                dl = d - lo
                m = (dl >= 0) & (dl < NPT)
                pc = plsc.all_reduce_population_count(m)
                pc0 = pc[0]

                @pl.when(pc0 > 0)
                def _():
                    inc = jnp.where(m, ones, zeros)
                    pos = off + plsc.cumsum(inc) - inc
                    plsc.store_scatter(posv, [pos], arange + i * 16, mask=m)

                return off + pc0

            msel = lax.fori_loop(0, CHUNK // 16, select, 0)
            plsc.store_scatter(posv, [msel + arange], zeros)

            def expand(i2, _):
                sl = pl.ds(i2 * 16, 16)
                p = posv[sl]
                sv = plsc.load_gather(srcv, [p])
                tv = plsc.load_gather(etv, [p])
                dd = plsc.load_gather(dstv, [p])
                flatv[sl] = tv * NP + sv
                dstlv[sl] = dd - lo
                return 0

            lax.fori_loop(0, (msel + 15) // 16, expand, 0)

            for g in range(GB // 16):
                pos = msel + arange + g * 16
                plsc.store_scatter(flatv, [pos], zeros)
                plsc.store_scatter(dstlv, [pos], trash)

            nb = (msel + GB - 1) // GB

            def gather_block(b, _):
                pltpu.async_copy(
                    h_hbm.at[flatv.at[pl.ds(b * GB, GB)]], rows, sem).wait()

                def group_body(g, _):
                    ebase = b * GB + g * 16
                    dlv16 = dstlv[pl.ds(ebase, 16)]
                    for jj in range(16):
                        sd = dlv16[jj] * D
                        for k in range(D // 16):
                            slk = pl.ds(sd + k * 16, 16)
                            rv = rows[g * 16 + jj, pl.ds(k * 16, 16)]
                            if is_max:
                                acc[slk] = jnp.maximum(acc[slk], rv)
                            else:
                                acc[slk] = acc[slk] + rv
                    return 0

                ng = jnp.minimum(GB // 16, (msel - b * GB + 15) // 16)
                lax.fori_loop(0, ng, group_body, 0)
                return 0

            lax.fori_loop(0, nb, gather_block, 0)
            return 0

        lax.fori_loop(0, NCHUNK, chunk_body, 0)

        pltpu.sync_copy(acc.at[pl.ds(0, NPT * D)],
                        out_hbm.at[pl.ds(wid * NPT * D, NPT * D)])

    return agg_kernel


_sc_agg_max = _make_sc_agg(True)

GB2 = 80
EPT = E // NW
EPTP = 10240
CH2 = 2560
NBC = CH2 // GB2
NCH2 = EPTP // CH2
NTRASH = 16
RPS = NP // 16


def _make_sc_sum():
    mesh = plsc.VectorSubcoreMesh(core_axis_name="c", subcore_axis_name="s")

    @functools.partial(
        pl.kernel,
        mesh=mesh,
        compiler_params=pltpu.CompilerParams(needs_layout_passes=False),
        out_type=jax.ShapeDtypeStruct((2 * NP, D), jnp.float32),
        scratch_types=[
            pltpu.VMEM((CH2,), jnp.int32),
            pltpu.VMEM((CH2,), jnp.int32),
            pltpu.VMEM((CH2,), jnp.int32),
            pltpu.VMEM((NBC, GB2), jnp.int32),
            pltpu.VMEM((GB2, D), jnp.float32),
            pltpu.VMEM((GB2, D), jnp.float32),
            pltpu.VMEM((128, D), jnp.float32),
            pltpu.VMEM_SHARED((NP + NTRASH, D), jnp.float32),
            pltpu.SemaphoreType.DMA,
            pltpu.SemaphoreType.DMA,
            pltpu.SemaphoreType.DMA,
        ],
    )
    def sum_kernel(h_hbm, src_hbm, et_hbm, dst2_hbm, out_hbm,
                   srcv, etv, idxv, dst2, rows0, rows1, zblk, shacc,
                   gsem0, gsem1, ssem):
        cid = lax.axis_index("c")
        sid = lax.axis_index("s")
        wid = sid * 2 + cid

        zero16 = jnp.zeros((16,), jnp.float32)

        def zrow(i, _):
            for k in range(D // 16):
                zblk[i, pl.ds(k * 16, 16)] = zero16
            return 0

        lax.fori_loop(0, 128, zrow, 0)

        for z in range(RPS // 128):
            pltpu.sync_copy(zblk,
                            shacc.at[pl.ds(sid * RPS + z * 128, 128)])

        plsc.subcore_barrier()

        rbufs = (rows0, rows1)
        gsems = (gsem0, gsem1)

        def chunk_body(c, _):
            ebase = wid * EPTP + c * CH2
            pltpu.sync_copy(src_hbm.at[pl.ds(ebase, CH2)], srcv)
            pltpu.sync_copy(et_hbm.at[pl.ds(ebase, CH2)], etv)
            pltpu.sync_copy(
                dst2_hbm.at[pl.ds(wid * (EPTP // GB2) + c * NBC, NBC)], dst2)

            def flatstep(i, _):
                sl = pl.ds(i * 16, 16)
                idxv[sl] = etv[sl] * NP + srcv[sl]
                return 0

            lax.fori_loop(0, CH2 // 16, flatstep, 0)

            cps = [None, None]
            cps[0] = pltpu.async_copy(
                h_hbm.at[idxv.at[pl.ds(0, GB2)]], rbufs[0], gsems[0])
            for b in range(NBC):
                cur = b % 2
                cps[cur].wait()
                if b + 1 < NBC:
                    nxt = (b + 1) % 2
                    cps[nxt] = pltpu.async_copy(
                        h_hbm.at[idxv.at[pl.ds((b + 1) * GB2, GB2)]],
                        rbufs[nxt], gsems[nxt])
                pltpu.async_copy(rbufs[cur], shacc.at[dst2.at[b]], ssem,
                                 add=True).wait()
            return 0

        lax.fori_loop(0, NCH2, chunk_body, 0)

        plsc.subcore_barrier()

        for z in range(RPS // 128):
            base = sid * RPS + z * 128
            pltpu.sync_copy(shacc.at[pl.ds(base, 128)],
                            out_hbm.at[pl.ds(cid * NP + base, 128)])

    return sum_kernel


_sc_agg_sum = _make_sc_sum()


def kernel(x, edge_index, edge_type, c1_basis, c1_comp, c1_root, c1_bias,
           ln1_g, ln1_b, c2_basis, c2_comp, c2_root, c2_bias, ln2_g, ln2_b,
           w1, b1, w2, b2, w3, b3):
    xp = jnp.pad(x, ((0, NP - N), (0, 0)))
    src = edge_index[0]
    dst = edge_index[1]

    w1r = _assemble_w(c1_comp, c1_basis, D, D)
    h1 = _relation_matmul(xp, w1r).reshape(R * NP, D)
    agg1 = _sc_agg_max(h1, src, dst, edge_type).reshape(NP, D)
    x1 = _mid_layer(agg1, xp, c1_root, c1_bias, ln1_g, ln1_b)

    w2r = _assemble_w(c2_comp, c2_basis, D, D)
    h2 = _relation_matmul(x1, w2r).reshape(R * NP, D)
    pad = ((0, 0), (0, EPTP - EPT))
    srcp = jnp.pad(src.reshape(NW, EPT), pad).reshape(-1)
    etp = jnp.pad(edge_type.reshape(NW, EPT), pad).reshape(-1)
    dstp = jnp.pad(dst.reshape(NW, EPT), pad,
                   constant_values=NP).reshape(NW * EPTP // GB2, GB2)
    agg2 = _sc_agg_sum(h2, srcp, etp, dstp).reshape(2, NP, D)
    out = _tail_layer(agg2[0], agg2[1], x1, c2_root, c2_bias, ln2_g, ln2_b,
                      w1, b1, w2, b2, w3, b3)
    return out[:N]


# speedup vs baseline: 1.0107x; 1.0107x over previous
import functools

import jax
import jax.numpy as jnp
from jax import lax
from jax.experimental import pallas as pl
from jax.experimental.pallas import tpu as pltpu
from jax.experimental.pallas import tpu_sc as plsc

N = 10000
NP = 10240
E = 320000
D = 128
R = 9
NB = 6

NW = 32
NPT = NP // NW
CHUNK = 8000
NCHUNK = E // CHUNK
GB = 128
BN = 512
NBLK = NP // BN


def _w_body(comp_ref, basis_ref, out_ref):
    r = pl.program_id(0)
    onehot = (lax.broadcasted_iota(jnp.int32, (R, 1), 0) == r).astype(jnp.float32)
    cw = jnp.sum(comp_ref[...] * onehot, axis=0)
    out_ref[0] = jnp.sum(basis_ref[...] * cw[:, None, None], axis=0)


def _assemble_w(comp, basis, d_in, d_out):
    return pl.pallas_call(
        _w_body,
        grid=(R,),
        in_specs=[
            pl.BlockSpec((R, NB), lambda r: (0, 0)),
            pl.BlockSpec((NB, d_in, d_out), lambda r: (0, 0, 0)),
        ],
        out_specs=pl.BlockSpec((1, d_in, d_out), lambda r: (r, 0, 0)),
        out_shape=jax.ShapeDtypeStruct((R, d_in, d_out), jnp.float32),
    )(comp, basis)


def _h_body(x_ref, w_ref, out_ref):
    out_ref[0] = jnp.dot(x_ref[...], w_ref[0],
                         preferred_element_type=jnp.float32)


def _relation_matmul(x, w):
    d_in, d_out = w.shape[1], w.shape[2]
    return pl.pallas_call(
        _h_body,
        grid=(R, NBLK),
        in_specs=[
            pl.BlockSpec((BN, d_in), lambda r, i: (i, 0)),
            pl.BlockSpec((1, d_in, d_out), lambda r, i: (r, 0, 0)),
        ],
        out_specs=pl.BlockSpec((1, BN, d_out), lambda r, i: (r, i, 0)),
        out_shape=jax.ShapeDtypeStruct((R, NP, d_out), jnp.float32),
    )(x, w)


def _ln(h, g, b):
    mu = jnp.mean(h, axis=-1, keepdims=True)
    var = jnp.mean((h - mu) ** 2, axis=-1, keepdims=True)
    return (h - mu) / jnp.sqrt(var + 1e-5) * g + b


def _mid_body(agg_ref, x_ref, root_ref, bias_ref, g_ref, b_ref, out_ref):
    a = agg_ref[...]
    a = jnp.where(jnp.isfinite(a), a, 0.0)
    h = a + jnp.dot(x_ref[...], root_ref[...],
                    preferred_element_type=jnp.float32) + bias_ref[0][None, :]
    h = _ln(h, g_ref[0][None, :], b_ref[0][None, :])
    out_ref[...] = jnp.maximum(h, 0.0)


def _mid_layer(agg, x, root, bias, g, b):
    vec = lambda i: (i, 0)
    return pl.pallas_call(
        _mid_body,
        grid=(NBLK,),
        in_specs=[
            pl.BlockSpec((BN, D), vec),
            pl.BlockSpec((BN, D), vec),
            pl.BlockSpec((D, D), lambda i: (0, 0)),
            pl.BlockSpec((1, D), lambda i: (0, 0)),
            pl.BlockSpec((1, D), lambda i: (0, 0)),
            pl.BlockSpec((1, D), lambda i: (0, 0)),
        ],
        out_specs=pl.BlockSpec((BN, D), vec),
        out_shape=jax.ShapeDtypeStruct((NP, D), jnp.float32),
    )(agg, x, root, bias.reshape(1, D), g.reshape(1, D), b.reshape(1, D))


def _tail_body(agg_ref, aggb_ref, x1_ref, root_ref, bias_ref, g_ref, b_ref,
               w1_ref, b1_ref, w2_ref, b2_ref, w3_ref, b3_ref, out_ref):
    x1 = x1_ref[...]
    h = agg_ref[...] + aggb_ref[...] + jnp.dot(
        x1, root_ref[...], preferred_element_type=jnp.float32)
    h = h + bias_ref[0][None, :]
    h = _ln(h, g_ref[0][None, :], b_ref[0][None, :])
    h = jnp.maximum(h, 0.0)
    h = h + x1 * 0.5
    h = jax.nn.gelu(jnp.dot(h, w1_ref[...],
                            preferred_element_type=jnp.float32)
                    + b1_ref[0][None, :])
    h = jax.nn.gelu(jnp.dot(h, w2_ref[...],
                            preferred_element_type=jnp.float32)
                    + b2_ref[0][None, :])
    o = jnp.dot(h, w3_ref[...], preferred_element_type=jnp.float32) \
        + b3_ref[0][None, :]
    m = jnp.max(o, axis=-1, keepdims=True)
    lse = m + jnp.log(jnp.sum(jnp.exp(o - m), axis=-1, keepdims=True))
    out_ref[...] = o - lse


def _tail_layer(agg, aggb, x1, root, bias, g, b, w1, b1, w2, b2, w3, b3):
    vec = lambda i: (i, 0)
    fix = lambda i: (0, 0)
    dh2 = D // 2
    do = 64
    return pl.pallas_call(
        _tail_body,
        grid=(NBLK,),
        in_specs=[
            pl.BlockSpec((BN, D), vec),
            pl.BlockSpec((BN, D), vec),
            pl.BlockSpec((BN, D), vec),
            pl.BlockSpec((D, D), fix),
            pl.BlockSpec((1, D), fix),
            pl.BlockSpec((1, D), fix),
            pl.BlockSpec((1, D), fix),
            pl.BlockSpec((D, D), fix),
            pl.BlockSpec((1, D), fix),
            pl.BlockSpec((D, dh2), fix),
            pl.BlockSpec((1, dh2), fix),
            pl.BlockSpec((dh2, do), fix),
            pl.BlockSpec((1, do), fix),
        ],
        out_specs=pl.BlockSpec((BN, do), vec),
        out_shape=jax.ShapeDtypeStruct((NP, do), jnp.float32),
    )(agg, aggb, x1, root, bias.reshape(1, D), g.reshape(1, D),
      b.reshape(1, D), w1, b1.reshape(1, D), w2, b2.reshape(1, dh2),
      w3, b3.reshape(1, do))


def _make_sc_agg(is_max):
    mesh = plsc.VectorSubcoreMesh(core_axis_name="c", subcore_axis_name="s")
    init_val = -jnp.inf if is_max else 0.0

    @functools.partial(
        pl.kernel,
        mesh=mesh,
        compiler_params=pltpu.CompilerParams(needs_layout_passes=False),
        out_type=jax.ShapeDtypeStruct((NP * D,), jnp.float32),
        scratch_types=[
            pltpu.VMEM((CHUNK,), jnp.int32),
            pltpu.VMEM((CHUNK,), jnp.int32),
            pltpu.VMEM((CHUNK,), jnp.int32),
            pltpu.VMEM((CHUNK + GB,), jnp.int32),
            pltpu.VMEM((CHUNK + GB,), jnp.int32),
            pltpu.VMEM((GB, D), jnp.float32),
            pltpu.VMEM((GB, D), jnp.float32),
            pltpu.VMEM(((NPT + 1) * D,), jnp.float32),
            pltpu.VMEM((CHUNK + GB,), jnp.int32),
            pltpu.SemaphoreType.DMA,
            pltpu.SemaphoreType.DMA,
        ],
    )
    def agg_kernel(h_hbm, src_hbm, dst_hbm, et_hbm, out_hbm,
                   srcv, dstv, etv, flatv, dstlv, rows0, rows1, acc, posv,
                   sem0, sem1):
        wid = lax.axis_index("s") * 2 + lax.axis_index("c")
        lo = wid * NPT

        fill = jnp.full((16,), init_val, jnp.float32)
        arange = jnp.arange(16, dtype=jnp.int32)
        ones = jnp.ones((16,), jnp.int32)
        zeros = jnp.zeros((16,), jnp.int32)
        trash = jnp.full((16,), NPT, jnp.int32)

        def init_row(i, _):
            acc[pl.ds(i * 16, 16)] = fill
            return 0

        lax.fori_loop(0, (NPT + 1) * D // 16, init_row, 0)

        def chunk_body(c, _):
            base = c * CHUNK
            pltpu.sync_copy(src_hbm.at[pl.ds(base, CHUNK)], srcv)
            pltpu.sync_copy(dst_hbm.at[pl.ds(base, CHUNK)], dstv)
            pltpu.sync_copy(et_hbm.at[pl.ds(base, CHUNK)], etv)

            def select(i, off_v):
                sl = pl.ds(i * 16, 16)
                d = dstv[sl]
                dl = d - lo
                m = (dl >= 0) & (dl < NPT)
                pc = plsc.all_reduce_population_count(m)
                inc = jnp.where(m, ones, zeros)
                pos = off_v + plsc.cumsum(inc) - inc
                plsc.store_scatter(posv, [pos], arange + i * 16, mask=m)
                return off_v + pc

            off_v = lax.fori_loop(0, CHUNK // 16, select, zeros)
            msel = off_v[0]
            plsc.store_scatter(posv, [msel + arange], zeros)

            def expand(i2, _):
                sl = pl.ds(i2 * 16, 16)
                p = posv[sl]
                sv = plsc.load_gather(srcv, [p])
                tv = plsc.load_gather(etv, [p])
                dd = plsc.load_gather(dstv, [p])
                flatv[sl] = tv * NP + sv
                dstlv[sl] = dd - lo
                return 0

            lax.fori_loop(0, (msel + 15) // 16, expand, 0)

            for g in range(GB // 16):
                pos = msel + arange + g * 16
                plsc.store_scatter(flatv, [pos], zeros)
                plsc.store_scatter(dstlv, [pos], trash)

            nb = (msel + GB - 1) // GB

            @pl.when(nb > 0)
            def _():
                pltpu.async_copy(
                    h_hbm.at[flatv.at[pl.ds(0, GB)]], rows0, sem0)

            def gather_block(b, _):
                def make_body(rbuf, csem, obuf, osem):
                    def body():
                        pltpu.make_async_copy(
                            h_hbm.at[flatv.at[pl.ds(b * GB, GB)]],
                            rbuf, csem).wait()

                        @pl.when(b + 1 < nb)
                        def _():
                            pltpu.async_copy(
                                h_hbm.at[flatv.at[pl.ds((b + 1) * GB, GB)]],
                                obuf, osem)

                        def group_body(g, _):
                            ebase = b * GB + g * 16
                            dlv16 = dstlv[pl.ds(ebase, 16)]
                            for jj in range(16):
                                sd = dlv16[jj] * D
                                for k in range(D // 16):
                                    slk = pl.ds(sd + k * 16, 16)
                                    rv = rbuf[g * 16 + jj, pl.ds(k * 16, 16)]
                                    if is_max:
                                        acc[slk] = jnp.maximum(acc[slk], rv)
                                    else:
                                        acc[slk] = acc[slk] + rv
                            return 0

                        ng = jnp.minimum(GB // 16,
                                         (msel - b * GB + 15) // 16)
                        lax.fori_loop(0, ng, group_body, 0)
                    return body

                even = (b % 2) == 0
                pl.when(even)(make_body(rows0, sem0, rows1, sem1))
                pl.when(jnp.logical_not(even))(
                    make_body(rows1, sem1, rows0, sem0))
                return 0

            lax.fori_loop(0, nb, gather_block, 0)
            return 0

        lax.fori_loop(0, NCHUNK, chunk_body, 0)

        pltpu.sync_copy(acc.at[pl.ds(0, NPT * D)],
                        out_hbm.at[pl.ds(wid * NPT * D, NPT * D)])

    return agg_kernel


_sc_agg_max = _make_sc_agg(True)

GB2 = 80
EPT = E // NW
EPTP = 10240
CH2 = 2560
NBC = CH2 // GB2
NCH2 = EPTP // CH2
NTRASH = 16
RPS = NP // 16


def _make_sc_sum():
    mesh = plsc.VectorSubcoreMesh(core_axis_name="c", subcore_axis_name="s")

    @functools.partial(
        pl.kernel,
        mesh=mesh,
        compiler_params=pltpu.CompilerParams(needs_layout_passes=False),
        out_type=jax.ShapeDtypeStruct((2 * NP, D), jnp.float32),
        scratch_types=[
            pltpu.VMEM((CH2,), jnp.int32),
            pltpu.VMEM((CH2,), jnp.int32),
            pltpu.VMEM((CH2,), jnp.int32),
            pltpu.VMEM((NBC, GB2), jnp.int32),
            pltpu.VMEM((GB2, D), jnp.float32),
            pltpu.VMEM((GB2, D), jnp.float32),
            pltpu.VMEM((128, D), jnp.float32),
            pltpu.VMEM_SHARED((NP + NTRASH, D), jnp.float32),
            pltpu.SemaphoreType.DMA,
            pltpu.SemaphoreType.DMA,
            pltpu.SemaphoreType.DMA,
        ],
    )
    def sum_kernel(h_hbm, src_hbm, et_hbm, dst2_hbm, out_hbm,
                   srcv, etv, idxv, dst2, rows0, rows1, zblk, shacc,
                   gsem0, gsem1, ssem):
        cid = lax.axis_index("c")
        sid = lax.axis_index("s")
        wid = sid * 2 + cid

        zero16 = jnp.zeros((16,), jnp.float32)

        def zrow(i, _):
            for k in range(D // 16):
                zblk[i, pl.ds(k * 16, 16)] = zero16
            return 0

        lax.fori_loop(0, 128, zrow, 0)

        for z in range(RPS // 128):
            pltpu.sync_copy(zblk,
                            shacc.at[pl.ds(sid * RPS + z * 128, 128)])

        plsc.subcore_barrier()

        rbufs = (rows0, rows1)
        gsems = (gsem0, gsem1)

        def chunk_body(c, _):
            ebase = wid * EPTP + c * CH2
            pltpu.sync_copy(src_hbm.at[pl.ds(ebase, CH2)], srcv)
            pltpu.sync_copy(et_hbm.at[pl.ds(ebase, CH2)], etv)
            pltpu.sync_copy(
                dst2_hbm.at[pl.ds(wid * (EPTP // GB2) + c * NBC, NBC)], dst2)

            def flatstep(i, _):
                sl = pl.ds(i * 16, 16)
                idxv[sl] = etv[sl] * NP + srcv[sl]
                return 0

            lax.fori_loop(0, CH2 // 16, flatstep, 0)

            cps = [None, None]
            cps[0] = pltpu.async_copy(
                h_hbm.at[idxv.at[pl.ds(0, GB2)]], rbufs[0], gsems[0])
            for b in range(NBC):
                cur = b % 2
                cps[cur].wait()
                if b + 1 < NBC:
                    nxt = (b + 1) % 2
                    cps[nxt] = pltpu.async_copy(
                        h_hbm.at[idxv.at[pl.ds((b + 1) * GB2, GB2)]],
                        rbufs[nxt], gsems[nxt])
                pltpu.async_copy(rbufs[cur], shacc.at[dst2.at[b]], ssem,
                                 add=True).wait()
            return 0

        lax.fori_loop(0, NCH2, chunk_body, 0)

        plsc.subcore_barrier()

        for z in range(RPS // 128):
            base = sid * RPS + z * 128
            pltpu.sync_copy(shacc.at[pl.ds(base, 128)],
                            out_hbm.at[pl.ds(cid * NP + base, 128)])

    return sum_kernel


_sc_agg_sum = _make_sc_sum()


def kernel(x, edge_index, edge_type, c1_basis, c1_comp, c1_root, c1_bias,
           ln1_g, ln1_b, c2_basis, c2_comp, c2_root, c2_bias, ln2_g, ln2_b,
           w1, b1, w2, b2, w3, b3):
    xp = jnp.pad(x, ((0, NP - N), (0, 0)))
    src = edge_index[0]
    dst = edge_index[1]

    w1r = _assemble_w(c1_comp, c1_basis, D, D)
    h1 = _relation_matmul(xp, w1r).reshape(R * NP, D)
    agg1 = _sc_agg_max(h1, src, dst, edge_type).reshape(NP, D)
    x1 = _mid_layer(agg1, xp, c1_root, c1_bias, ln1_g, ln1_b)

    w2r = _assemble_w(c2_comp, c2_basis, D, D)
    h2 = _relation_matmul(x1, w2r).reshape(R * NP, D)
    pad = ((0, 0), (0, EPTP - EPT))
    srcp = jnp.pad(src.reshape(NW, EPT), pad).reshape(-1)
    etp = jnp.pad(edge_type.reshape(NW, EPT), pad).reshape(-1)
    dstp = jnp.pad(dst.reshape(NW, EPT), pad,
                   constant_values=NP).reshape(NW * EPTP // GB2, GB2)
    agg2 = _sc_agg_sum(h2, srcp, etp, dstp).reshape(2, NP, D)
    out = _tail_layer(agg2[0], agg2[1], x1, c2_root, c2_bias, ln2_g, ln2_b,
                      w1, b1, w2, b2, w3, b3)
    return out[:N]
